# Initial kernel scaffold; baseline (speedup 1.0000x reference)
import jax
import jax.numpy as jnp
from jax.experimental import pallas as pl


def kernel(x, edge_index, edge_attr, batch, params):
    raise NotImplementedError("write your pallas kernel here")


# speedup vs baseline: 3.1587x; 3.1587x over previous
import functools

import jax
import jax.numpy as jnp
from jax import lax
from jax.experimental import pallas as pl
from jax.experimental.pallas import tpu as pltpu
from jax.experimental.pallas import tpu_sc as plsc

N = 10000
F = 16
HID = 25
D_EDGE = 3
C_OUT = 10
NUM_GRAPHS = 16
E = 160000

NC, NS = 2, 16
NW = NC * NS
EPT = 5120
E_PAD = NW * EPT
CHUNK = 128
NCH = EPT // CHUNK
N_ACC = 10016
RPT = N_ACC // NS

BE = 2048
BN = 2000


def _sc_mesh():
    return plsc.VectorSubcoreMesh(core_axis_name="c", subcore_axis_name="s",
                                  num_cores=NC, num_subcores=NS)


def _sc_params():
    return pltpu.CompilerParams(use_tc_tiling_on_sc=False)


def _gather_body(table_hbm, idx_hbm, out_hbm, idx_v, rows_v, sem):
    wid = lax.axis_index("c") * NS + lax.axis_index("s")
    pltpu.sync_copy(idx_hbm.at[pl.ds(wid * NCH, NCH)], idx_v)

    def fire(j, carry):
        pltpu.async_copy(table_hbm.at[idx_v.at[j]],
                         rows_v.at[pl.ds(j * CHUNK, CHUNK)], sem)
        return carry

    lax.fori_loop(0, NCH, fire, 0)

    def drain(j, carry):
        pltpu.make_async_copy(table_hbm.at[idx_v.at[j]],
                              rows_v.at[pl.ds(j * CHUNK, CHUNK)], sem).wait()
        return carry

    lax.fori_loop(0, NCH, drain, 0)
    pltpu.sync_copy(rows_v, out_hbm.at[pl.ds(wid * EPT, EPT)])


@functools.cache
def _sc_gather():
    return pl.kernel(
        _gather_body,
        out_type=jax.ShapeDtypeStruct((E_PAD, F), jnp.float32),
        mesh=_sc_mesh(),
        compiler_params=_sc_params(),
        scratch_types=[
            pltpu.VMEM((NCH, CHUNK), jnp.int32),
            pltpu.VMEM((EPT, F), jnp.float32),
            pltpu.SemaphoreType.DMA,
        ],
    )


def _scatter_counts_body(msg_hbm, idx_hbm, s_out, c_out,
                         idx_v, msg_v, zbuf, acc_s, ones_v, acc_c, sem):
    cid = lax.axis_index("c")
    sid = lax.axis_index("s")
    wid = cid * NS + sid

    def zfill(i, carry):
        zbuf[i] = jnp.zeros((F,), jnp.float32)
        return carry

    lax.fori_loop(0, RPT, zfill, 0)

    def ofill(i, carry):
        ones_v[i] = jnp.ones((F,), jnp.float32)
        return carry

    lax.fori_loop(0, CHUNK, ofill, 0)
    pltpu.sync_copy(zbuf, acc_s.at[pl.ds(sid * RPT, RPT)])
    pltpu.sync_copy(zbuf, acc_c.at[pl.ds(sid * RPT, RPT)])
    pltpu.sync_copy(idx_hbm.at[pl.ds(wid * NCH, NCH)], idx_v)
    pltpu.sync_copy(msg_hbm.at[pl.ds(wid * EPT, EPT)], msg_v)
    plsc.subcore_barrier()

    def fire(j, carry):
        pltpu.async_copy(msg_v.at[pl.ds(j * CHUNK, CHUNK)],
                         acc_s.at[idx_v.at[j]], sem, add=True)
        return carry

    lax.fori_loop(0, NCH, fire, 0)

    def drain(j, carry):
        pltpu.make_async_copy(msg_v.at[pl.ds(j * CHUNK, CHUNK)],
                              acc_s.at[idx_v.at[j]], sem).wait()
        return carry

    lax.fori_loop(0, NCH, drain, 0)

    def fire_c(j, carry):
        pltpu.async_copy(ones_v, acc_c.at[idx_v.at[j]], sem, add=True)
        return carry

    lax.fori_loop(0, NCH, fire_c, 0)

    def drain_c(j, carry):
        pltpu.make_async_copy(ones_v, acc_c.at[idx_v.at[j]], sem).wait()
        return carry

    lax.fori_loop(0, NCH, drain_c, 0)
    plsc.subcore_barrier()
    pltpu.sync_copy(acc_s.at[pl.ds(sid * RPT, RPT)],
                    s_out.at[cid, pl.ds(sid * RPT, RPT)])
    pltpu.sync_copy(acc_c.at[pl.ds(sid * RPT, RPT)],
                    c_out.at[cid, pl.ds(sid * RPT, RPT)])


@functools.cache
def _sc_scatter_counts():
    return pl.kernel(
        _scatter_counts_body,
        out_type=(jax.ShapeDtypeStruct((NC, N_ACC, F), jnp.float32),
                  jax.ShapeDtypeStruct((NC, N_ACC, F), jnp.float32)),
        mesh=_sc_mesh(),
        compiler_params=_sc_params(),
        scratch_types=[
            pltpu.VMEM((NCH, CHUNK), jnp.int32),
            pltpu.VMEM((EPT, F), jnp.float32),
            pltpu.VMEM((RPT, F), jnp.float32),
            pltpu.VMEM_SHARED((N_ACC, F), jnp.float32),
            pltpu.VMEM((CHUNK, F), jnp.float32),
            pltpu.VMEM_SHARED((N_ACC, F), jnp.float32),
            pltpu.SemaphoreType.DMA,
        ],
    )


def _scatter_body(msg_hbm, idx_hbm, s_out, idx_v, msg_v, zbuf, acc_s, sem):
    cid = lax.axis_index("c")
    sid = lax.axis_index("s")
    wid = cid * NS + sid

    def zfill(i, carry):
        zbuf[i] = jnp.zeros((F,), jnp.float32)
        return carry

    lax.fori_loop(0, RPT, zfill, 0)
    pltpu.sync_copy(zbuf, acc_s.at[pl.ds(sid * RPT, RPT)])
    pltpu.sync_copy(idx_hbm.at[pl.ds(wid * NCH, NCH)], idx_v)
    pltpu.sync_copy(msg_hbm.at[pl.ds(wid * EPT, EPT)], msg_v)
    plsc.subcore_barrier()

    def fire(j, carry):
        pltpu.async_copy(msg_v.at[pl.ds(j * CHUNK, CHUNK)],
                         acc_s.at[idx_v.at[j]], sem, add=True)
        return carry

    lax.fori_loop(0, NCH, fire, 0)

    def drain(j, carry):
        pltpu.make_async_copy(msg_v.at[pl.ds(j * CHUNK, CHUNK)],
                              acc_s.at[idx_v.at[j]], sem).wait()
        return carry

    lax.fori_loop(0, NCH, drain, 0)
    plsc.subcore_barrier()
    pltpu.sync_copy(acc_s.at[pl.ds(sid * RPT, RPT)],
                    s_out.at[cid, pl.ds(sid * RPT, RPT)])


@functools.cache
def _sc_scatter():
    return pl.kernel(
        _scatter_body,
        out_type=jax.ShapeDtypeStruct((NC, N_ACC, F), jnp.float32),
        mesh=_sc_mesh(),
        compiler_params=_sc_params(),
        scratch_types=[
            pltpu.VMEM((NCH, CHUNK), jnp.int32),
            pltpu.VMEM((EPT, F), jnp.float32),
            pltpu.VMEM((RPT, F), jnp.float32),
            pltpu.VMEM_SHARED((N_ACC, F), jnp.float32),
            pltpu.SemaphoreType.DMA,
        ],
    )


def _msg_body(ea_ref, xs_ref, w1_ref, b1_ref, w2_ref, e1_ref, r_ref, out_ref):
    h = jnp.dot(ea_ref[...], w1_ref[...], preferred_element_type=jnp.float32)
    h = jnp.maximum(h + b1_ref[...], 0.0)
    w = jnp.dot(h.astype(jnp.bfloat16), w2_ref[...],
                preferred_element_type=jnp.float32)
    xe = jnp.dot(xs_ref[...].astype(jnp.bfloat16), e1_ref[...],
                 preferred_element_type=jnp.float32)
    out_ref[...] = jnp.dot((w * xe).astype(jnp.bfloat16), r_ref[...],
                           preferred_element_type=jnp.float32)


def _tc_msg(ea, xs, w1e, b1e, w2e, e1, r):
    grid = (E_PAD // BE,)
    return pl.pallas_call(
        _msg_body,
        grid=grid,
        in_specs=[
            pl.BlockSpec((BE, 8), lambda i: (i, 0)),
            pl.BlockSpec((BE, F), lambda i: (i, 0)),
            pl.BlockSpec((8, 32), lambda i: (0, 0)),
            pl.BlockSpec((1, 32), lambda i: (0, 0)),
            pl.BlockSpec((32, 256), lambda i: (0, 0)),
            pl.BlockSpec((F, 256), lambda i: (0, 0)),
            pl.BlockSpec((256, F), lambda i: (0, 0)),
        ],
        out_specs=pl.BlockSpec((BE, F), lambda i: (i, 0)),
        out_shape=jax.ShapeDtypeStruct((E_PAD, F), jnp.float32),
    )(ea, xs, w1e, b1e, w2e, e1, r)


def _node_body(s_ref, c_ref, h_ref, wr_ref, br_ref, out_ref):
    s = s_ref[0] + s_ref[1]
    c = c_ref[0] + c_ref[1]
    agg = s / jnp.maximum(c, 1.0)
    t = agg + jnp.dot(h_ref[...], wr_ref[...],
                      preferred_element_type=jnp.float32) + br_ref[...]
    out_ref[...] = jnp.where(t > 0.0, t, jnp.exp(jnp.minimum(t, 0.0)) - 1.0)


def _tc_node(s_part, c_part, h_prev, wr, br):
    grid = (N // BN,)
    return pl.pallas_call(
        _node_body,
        grid=grid,
        in_specs=[
            pl.BlockSpec((NC, BN, F), lambda i: (0, i, 0)),
            pl.BlockSpec((NC, BN, F), lambda i: (0, i, 0)),
            pl.BlockSpec((BN, F), lambda i: (i, 0)),
            pl.BlockSpec((F, F), lambda i: (0, 0)),
            pl.BlockSpec((1, F), lambda i: (0, 0)),
        ],
        out_specs=pl.BlockSpec((BN, F), lambda i: (i, 0)),
        out_shape=jax.ShapeDtypeStruct((N, F), jnp.float32),
    )(s_part, c_part, h_prev, wr, br)


def _final_body(s_ref, c_ref, h_ref, wr_ref, br_ref, b_ref, wf_ref, bf_ref,
                out_ref, acc, cnt):
    i = pl.program_id(0)

    @pl.when(i == 0)
    def _():
        acc[...] = jnp.zeros_like(acc)
        cnt[...] = jnp.zeros_like(cnt)

    s = s_ref[0] + s_ref[1]
    c = c_ref[0] + c_ref[1]
    agg = s / jnp.maximum(c, 1.0)
    t = agg + jnp.dot(h_ref[...], wr_ref[...],
                      preferred_element_type=jnp.float32) + br_ref[...]
    h2 = jnp.where(t > 0.0, t, jnp.exp(jnp.minimum(t, 0.0)) - 1.0)
    gid = lax.broadcasted_iota(jnp.int32, (NUM_GRAPHS, BN), 0)
    oh = (gid == b_ref[0]).astype(jnp.float32)
    acc[...] += jnp.dot(oh, h2, preferred_element_type=jnp.float32)
    cnt[...] += jnp.dot(oh, jnp.ones((BN, F), jnp.float32),
                        preferred_element_type=jnp.float32)
    pooled = acc[...] / jnp.maximum(cnt[...], 1.0)
    out_ref[...] = jnp.dot(pooled, wf_ref[...],
                           preferred_element_type=jnp.float32) + bf_ref[...]


def _tc_final(s_part, c_part, h_prev, wr, br, batch3d, wf, bf):
    grid = (N // BN,)
    return pl.pallas_call(
        _final_body,
        grid=grid,
        in_specs=[
            pl.BlockSpec((NC, BN, F), lambda i: (0, i, 0)),
            pl.BlockSpec((NC, BN, F), lambda i: (0, i, 0)),
            pl.BlockSpec((BN, F), lambda i: (i, 0)),
            pl.BlockSpec((F, F), lambda i: (0, 0)),
            pl.BlockSpec((1, F), lambda i: (0, 0)),
            pl.BlockSpec((1, 1, BN), lambda i: (i, 0, 0)),
            pl.BlockSpec((F, C_OUT), lambda i: (0, 0)),
            pl.BlockSpec((1, C_OUT), lambda i: (0, 0)),
        ],
        out_specs=pl.BlockSpec((NUM_GRAPHS, C_OUT), lambda i: (0, 0)),
        out_shape=jax.ShapeDtypeStruct((NUM_GRAPHS, C_OUT), jnp.float32),
        scratch_shapes=[
            pltpu.VMEM((NUM_GRAPHS, F), jnp.float32),
            pltpu.VMEM((NUM_GRAPHS, F), jnp.float32),
        ],
    )(s_part, c_part, h_prev, wr, br, batch3d, wf, bf)


def _fold_layer(p):
    g = p['gamma'] / jnp.sqrt(1.0 + 1e-5)
    w1f = p['W1'] * g[None, :]
    b1f = p['b1'] * g + p['beta']
    w1e = jnp.zeros((8, 32), jnp.float32).at[:D_EDGE, :HID].set(w1f)
    b1e = jnp.zeros((1, 32), jnp.float32).at[0, :HID].set(b1f).at[0, HID].set(1.0)
    w2e = jnp.zeros((32, 256), jnp.float32).at[:HID].set(p['W2']).at[HID].set(p['b2'])
    return w1e, b1e, w2e.astype(jnp.bfloat16), p['Wr'], p['br'][None, :]


def kernel(x, edge_index, edge_attr, batch, params):
    src = edge_index[0]
    dst = edge_index[1]
    pad = E_PAD - E
    src2d = jnp.concatenate(
        [src, jnp.zeros((pad,), jnp.int32)]).reshape(NW * NCH, CHUNK)
    dst2d = jnp.concatenate(
        [dst, jnp.full((pad,), N, jnp.int32)]).reshape(NW * NCH, CHUNK)
    ea = jnp.zeros((E_PAD, 8), jnp.float32).at[:E, :D_EDGE].set(edge_attr)

    e1 = jnp.kron(jnp.eye(F, dtype=jnp.bfloat16),
                  jnp.ones((1, F), jnp.bfloat16))
    r = jnp.kron(jnp.ones((F, 1), jnp.bfloat16),
                 jnp.eye(F, dtype=jnp.bfloat16))

    w1e0, b1e0, w2e0, wr0, br0 = _fold_layer(params['conv0'])
    w1e1, b1e1, w2e1, wr1, br1 = _fold_layer(params['conv1'])

    xs0 = _sc_gather()(x, src2d)
    msg0 = _tc_msg(ea, xs0, w1e0, b1e0, w2e0, e1, r)
    s0, c0 = _sc_scatter_counts()(msg0, dst2d)
    h1 = _tc_node(s0, c0, x, wr0, br0)
    xs1 = _sc_gather()(h1, src2d)
    msg1 = _tc_msg(ea, xs1, w1e1, b1e1, w2e1, e1, r)
    s1 = _sc_scatter()(msg1, dst2d)
    return _tc_final(s1, c0, h1, wr1, br1,
                     batch.reshape(N // BN, 1, BN), params['Wf'],
                     params['bf'][None, :])


# speedup vs baseline: 4.7917x; 1.5170x over previous
import functools

import jax
import jax.numpy as jnp
from jax import lax
from jax.experimental import pallas as pl
from jax.experimental.pallas import tpu as pltpu
from jax.experimental.pallas import tpu_sc as plsc

N = 10000
F = 16
HID = 25
D_EDGE = 3
C_OUT = 10
NUM_GRAPHS = 16
E = 160000

NC, NS = 2, 16
NW = NC * NS
EPT = 5120
E_PAD = NW * EPT
CHUNK = 128
NCH = EPT // CHUNK
N_ACC = 10240
RPT = N_ACC // NS

BE = 2048
BN = 2048


def _sc_mesh():
    return plsc.VectorSubcoreMesh(core_axis_name="c", subcore_axis_name="s",
                                  num_cores=NC, num_subcores=NS)


def _sc_params():
    return pltpu.CompilerParams(use_tc_tiling_on_sc=False)


def _gather_body(table_hbm, idx_hbm, out_hbm, idx_v, rows_v, sem):
    wid = lax.axis_index("c") * NS + lax.axis_index("s")
    pltpu.sync_copy(idx_hbm.at[pl.ds(wid * NCH, NCH)], idx_v)

    def fire(j, carry):
        pltpu.async_copy(table_hbm.at[idx_v.at[j]],
                         rows_v.at[pl.ds(j * CHUNK, CHUNK)], sem)
        return carry

    lax.fori_loop(0, NCH, fire, 0)

    def drain(j, carry):
        pltpu.make_async_copy(table_hbm.at[idx_v.at[j]],
                              rows_v.at[pl.ds(j * CHUNK, CHUNK)], sem).wait()
        return carry

    lax.fori_loop(0, NCH, drain, 0)
    pltpu.sync_copy(rows_v, out_hbm.at[pl.ds(wid * EPT, EPT)])


@functools.cache
def _sc_gather():
    return pl.kernel(
        _gather_body,
        out_type=jax.ShapeDtypeStruct((E_PAD, F), jnp.float32),
        mesh=_sc_mesh(),
        compiler_params=_sc_params(),
        scratch_types=[
            pltpu.VMEM((NCH, CHUNK), jnp.int32),
            pltpu.VMEM((EPT, F), jnp.float32),
            pltpu.SemaphoreType.DMA,
        ],
    )


def _scatter_counts_body(msg_hbm, idx_hbm, s_out, c_out,
                         idx_v, msg_v, zbuf, acc_s, ones_v, acc_c, sem):
    cid = lax.axis_index("c")
    sid = lax.axis_index("s")
    wid = cid * NS + sid

    def zfill(i, carry):
        zbuf[i] = jnp.zeros((F,), jnp.float32)
        return carry

    lax.fori_loop(0, RPT, zfill, 0)

    def ofill(i, carry):
        ones_v[i] = jnp.ones((F,), jnp.float32)
        return carry

    lax.fori_loop(0, CHUNK, ofill, 0)
    pltpu.sync_copy(zbuf, acc_s.at[pl.ds(sid * RPT, RPT)])
    pltpu.sync_copy(zbuf, acc_c.at[pl.ds(sid * RPT, RPT)])
    pltpu.sync_copy(idx_hbm.at[pl.ds(wid * NCH, NCH)], idx_v)
    pltpu.sync_copy(msg_hbm.at[pl.ds(wid * EPT, EPT)], msg_v)
    plsc.subcore_barrier()

    def fire(j, carry):
        pltpu.async_copy(msg_v.at[pl.ds(j * CHUNK, CHUNK)],
                         acc_s.at[idx_v.at[j]], sem, add=True)
        return carry

    lax.fori_loop(0, NCH, fire, 0)

    def drain(j, carry):
        pltpu.make_async_copy(msg_v.at[pl.ds(j * CHUNK, CHUNK)],
                              acc_s.at[idx_v.at[j]], sem).wait()
        return carry

    lax.fori_loop(0, NCH, drain, 0)

    def fire_c(j, carry):
        pltpu.async_copy(ones_v, acc_c.at[idx_v.at[j]], sem, add=True)
        return carry

    lax.fori_loop(0, NCH, fire_c, 0)

    def drain_c(j, carry):
        pltpu.make_async_copy(ones_v, acc_c.at[idx_v.at[j]], sem).wait()
        return carry

    lax.fori_loop(0, NCH, drain_c, 0)
    plsc.subcore_barrier()
    pltpu.sync_copy(acc_s.at[pl.ds(sid * RPT, RPT)],
                    s_out.at[cid, pl.ds(sid * RPT, RPT)])
    pltpu.sync_copy(acc_c.at[pl.ds(sid * RPT, RPT)],
                    c_out.at[cid, pl.ds(sid * RPT, RPT)])


@functools.cache
def _sc_scatter_counts():
    return pl.kernel(
        _scatter_counts_body,
        out_type=(jax.ShapeDtypeStruct((NC, N_ACC, F), jnp.float32),
                  jax.ShapeDtypeStruct((NC, N_ACC, F), jnp.float32)),
        mesh=_sc_mesh(),
        compiler_params=_sc_params(),
        scratch_types=[
            pltpu.VMEM((NCH, CHUNK), jnp.int32),
            pltpu.VMEM((EPT, F), jnp.float32),
            pltpu.VMEM((RPT, F), jnp.float32),
            pltpu.VMEM_SHARED((N_ACC, F), jnp.float32),
            pltpu.VMEM((CHUNK, F), jnp.float32),
            pltpu.VMEM_SHARED((N_ACC, F), jnp.float32),
            pltpu.SemaphoreType.DMA,
        ],
    )


def _scatter_body(msg_hbm, idx_hbm, s_out, idx_v, msg_v, zbuf, acc_s, sem):
    cid = lax.axis_index("c")
    sid = lax.axis_index("s")
    wid = cid * NS + sid

    def zfill(i, carry):
        zbuf[i] = jnp.zeros((F,), jnp.float32)
        return carry

    lax.fori_loop(0, RPT, zfill, 0)
    pltpu.sync_copy(zbuf, acc_s.at[pl.ds(sid * RPT, RPT)])
    pltpu.sync_copy(idx_hbm.at[pl.ds(wid * NCH, NCH)], idx_v)
    pltpu.sync_copy(msg_hbm.at[pl.ds(wid * EPT, EPT)], msg_v)
    plsc.subcore_barrier()

    def fire(j, carry):
        pltpu.async_copy(msg_v.at[pl.ds(j * CHUNK, CHUNK)],
                         acc_s.at[idx_v.at[j]], sem, add=True)
        return carry

    lax.fori_loop(0, NCH, fire, 0)

    def drain(j, carry):
        pltpu.make_async_copy(msg_v.at[pl.ds(j * CHUNK, CHUNK)],
                              acc_s.at[idx_v.at[j]], sem).wait()
        return carry

    lax.fori_loop(0, NCH, drain, 0)
    plsc.subcore_barrier()
    pltpu.sync_copy(acc_s.at[pl.ds(sid * RPT, RPT)],
                    s_out.at[cid, pl.ds(sid * RPT, RPT)])


@functools.cache
def _sc_scatter():
    return pl.kernel(
        _scatter_body,
        out_type=jax.ShapeDtypeStruct((NC, N_ACC, F), jnp.float32),
        mesh=_sc_mesh(),
        compiler_params=_sc_params(),
        scratch_types=[
            pltpu.VMEM((NCH, CHUNK), jnp.int32),
            pltpu.VMEM((EPT, F), jnp.float32),
            pltpu.VMEM((RPT, F), jnp.float32),
            pltpu.VMEM_SHARED((N_ACC, F), jnp.float32),
            pltpu.SemaphoreType.DMA,
        ],
    )


def _msg_body(ea_ref, xs_ref, w1_ref, b1_ref, w2_ref, e1_ref, r_ref, out_ref):
    h = jnp.dot(ea_ref[...], w1_ref[...], preferred_element_type=jnp.float32)
    h = jnp.maximum(h + b1_ref[...], 0.0)
    w = jnp.dot(h.astype(jnp.bfloat16), w2_ref[...],
                preferred_element_type=jnp.float32)
    xe = jnp.dot(xs_ref[...].astype(jnp.bfloat16), e1_ref[...],
                 preferred_element_type=jnp.float32)
    out_ref[...] = jnp.dot((w * xe).astype(jnp.bfloat16), r_ref[...],
                           preferred_element_type=jnp.float32)


BR = BE // 8


def _tc_msg(ea_p, xs_p, w1bd, b1t, w2bd, e1bd, rbd):
    grid = (E_PAD // BE,)
    return pl.pallas_call(
        _msg_body,
        grid=grid,
        in_specs=[
            pl.BlockSpec((BR, 64), lambda i: (i, 0)),
            pl.BlockSpec((BR, 128), lambda i: (i, 0)),
            pl.BlockSpec((64, 256), lambda i: (0, 0)),
            pl.BlockSpec((1, 256), lambda i: (0, 0)),
            pl.BlockSpec((256, 2048), lambda i: (0, 0)),
            pl.BlockSpec((128, 2048), lambda i: (0, 0)),
            pl.BlockSpec((2048, 128), lambda i: (0, 0)),
        ],
        out_specs=pl.BlockSpec((BR, 128), lambda i: (i, 0)),
        out_shape=jax.ShapeDtypeStruct((E_PAD // 8, 128), jnp.float32),
    )(ea_p, xs_p, w1bd, b1t, w2bd, e1bd, rbd)


BNP = BN // 8


def _node_body(s_ref, c_ref, h_ref, wr_ref, br_ref, out_ref):
    s = s_ref[0] + s_ref[1]
    c = c_ref[0] + c_ref[1]
    agg = s / jnp.maximum(c, 1.0)
    t = agg + jnp.dot(h_ref[...], wr_ref[...],
                      preferred_element_type=jnp.float32) + br_ref[...]
    out_ref[...] = jnp.where(t > 0.0, t, jnp.exp(jnp.minimum(t, 0.0)) - 1.0)


def _tc_node(s_part, c_part, h_prev_p, wrbd, brt):
    grid = (N_ACC // BN,)
    return pl.pallas_call(
        _node_body,
        grid=grid,
        in_specs=[
            pl.BlockSpec((NC, BNP, 128), lambda i: (0, i, 0)),
            pl.BlockSpec((NC, BNP, 128), lambda i: (0, i, 0)),
            pl.BlockSpec((BNP, 128), lambda i: (i, 0)),
            pl.BlockSpec((128, 128), lambda i: (0, 0)),
            pl.BlockSpec((1, 128), lambda i: (0, 0)),
        ],
        out_specs=pl.BlockSpec((BNP, 128), lambda i: (i, 0)),
        out_shape=jax.ShapeDtypeStruct((N_ACC // 8, 128), jnp.float32),
    )(s_part, c_part, h_prev_p, wrbd, brt)


def _pool_body(h_ref, b_ref, wf_ref, bf_ref, out_ref, acc, cnt):
    i = pl.program_id(0)

    @pl.when(i == 0)
    def _():
        acc[...] = jnp.zeros_like(acc)
        cnt[...] = jnp.zeros_like(cnt)

    gid = lax.broadcasted_iota(jnp.int32, (NUM_GRAPHS, BN), 0)
    oh = (gid == b_ref[0]).astype(jnp.float32)
    acc[...] += jnp.dot(oh, h_ref[...], preferred_element_type=jnp.float32)
    cnt[...] += jnp.dot(oh, jnp.ones((BN, F), jnp.float32),
                        preferred_element_type=jnp.float32)
    pooled = acc[...] / jnp.maximum(cnt[...], 1.0)
    out_ref[...] = jnp.dot(pooled, wf_ref[...],
                           preferred_element_type=jnp.float32) + bf_ref[...]


def _tc_pool(h2, batch3d, wf, bf):
    grid = (N_ACC // BN,)
    return pl.pallas_call(
        _pool_body,
        grid=grid,
        in_specs=[
            pl.BlockSpec((BN, F), lambda i: (i, 0)),
            pl.BlockSpec((1, 1, BN), lambda i: (i, 0, 0)),
            pl.BlockSpec((F, C_OUT), lambda i: (0, 0)),
            pl.BlockSpec((1, C_OUT), lambda i: (0, 0)),
        ],
        out_specs=pl.BlockSpec((NUM_GRAPHS, C_OUT), lambda i: (0, 0)),
        out_shape=jax.ShapeDtypeStruct((NUM_GRAPHS, C_OUT), jnp.float32),
        scratch_shapes=[
            pltpu.VMEM((NUM_GRAPHS, F), jnp.float32),
            pltpu.VMEM((NUM_GRAPHS, F), jnp.float32),
        ],
    )(h2, batch3d, wf, bf)


def _fold_layer(p, ey8):
    g = p['gamma'] / jnp.sqrt(1.0 + 1e-5)
    w1f = p['W1'] * g[None, :]
    b1f = p['b1'] * g + p['beta']
    w1e = jnp.zeros((8, 32), jnp.float32).at[:D_EDGE, :HID].set(w1f)
    b1e = jnp.zeros((1, 32), jnp.float32).at[0, :HID].set(b1f).at[0, HID].set(1.0)
    w2e = jnp.zeros((32, 256), jnp.float32).at[:HID].set(p['W2']).at[HID].set(p['b2'])
    w1bd = jnp.kron(ey8, w1e)
    b1t = jnp.tile(b1e, (1, 8))
    w2bd = jnp.kron(ey8.astype(jnp.bfloat16),
                    w2e.astype(jnp.bfloat16))
    wrbd = jnp.kron(ey8, p['Wr'])
    brt = jnp.tile(p['br'][None, :], (1, 8))
    return w1bd, b1t, w2bd, wrbd, brt


def kernel(x, edge_index, edge_attr, batch, params):
    src = edge_index[0]
    dst = edge_index[1]
    pad = E_PAD - E
    src2d = jnp.concatenate(
        [src, jnp.zeros((pad,), jnp.int32)]).reshape(NW * NCH, CHUNK)
    dst2d = jnp.concatenate(
        [dst, jnp.full((pad,), N, jnp.int32)]).reshape(NW * NCH, CHUNK)
    ea_p = jnp.zeros((E_PAD, 8), jnp.float32).at[:E, :D_EDGE].set(
        edge_attr).reshape(E_PAD // 8, 64)

    ey8 = jnp.eye(8, dtype=jnp.float32)
    ey8b = jnp.eye(8, dtype=jnp.bfloat16)
    e1bd = jnp.kron(ey8b, jnp.kron(jnp.eye(F, dtype=jnp.bfloat16),
                                   jnp.ones((1, F), jnp.bfloat16)))
    rbd = jnp.kron(ey8b, jnp.kron(jnp.ones((F, 1), jnp.bfloat16),
                                  jnp.eye(F, dtype=jnp.bfloat16)))

    w1bd0, b1t0, w2bd0, wrbd0, brt0 = _fold_layer(params['conv0'], ey8)
    w1bd1, b1t1, w2bd1, wrbd1, brt1 = _fold_layer(params['conv1'], ey8)

    x_p = jnp.zeros((N_ACC, F), jnp.float32).at[:N].set(x).reshape(
        N_ACC // 8, 128)
    batch3d = jnp.concatenate(
        [batch, jnp.full((N_ACC - N,), NUM_GRAPHS, jnp.int32)]).reshape(
        N_ACC // BN, 1, BN)

    xs0 = _sc_gather()(x, src2d)
    msg0 = _tc_msg(ea_p, xs0.reshape(E_PAD // 8, 128),
                   w1bd0, b1t0, w2bd0, e1bd, rbd)
    s0, c0 = _sc_scatter_counts()(msg0.reshape(E_PAD, F), dst2d)
    s0_p = s0.reshape(NC, N_ACC // 8, 128)
    c0_p = c0.reshape(NC, N_ACC // 8, 128)
    h1_p = _tc_node(s0_p, c0_p, x_p, wrbd0, brt0)
    h1 = h1_p.reshape(N_ACC, F)
    xs1 = _sc_gather()(h1, src2d)
    msg1 = _tc_msg(ea_p, xs1.reshape(E_PAD // 8, 128),
                   w1bd1, b1t1, w2bd1, e1bd, rbd)
    s1 = _sc_scatter()(msg1.reshape(E_PAD, F), dst2d)
    h2_p = _tc_node(s1.reshape(NC, N_ACC // 8, 128), c0_p, h1_p, wrbd1, brt1)
    return _tc_pool(h2_p.reshape(N_ACC, F), batch3d,
                    params['Wf'], params['bf'][None, :])


# speedup vs baseline: 4.8216x; 1.0062x over previous
import functools

import jax
import jax.numpy as jnp
from jax import lax
from jax.experimental import pallas as pl
from jax.experimental.pallas import tpu as pltpu
from jax.experimental.pallas import tpu_sc as plsc

N = 10000
F = 16
HID = 25
D_EDGE = 3
C_OUT = 10
NUM_GRAPHS = 16
E = 160000

NC, NS = 2, 16
NW = NC * NS
EPT = 5120
E_PAD = NW * EPT
CHUNK = 128
NCH = EPT // CHUNK
N_ACC = 10240
RPT = N_ACC // NS

BE = 2048
BN = 2048


def _sc_mesh():
    return plsc.VectorSubcoreMesh(core_axis_name="c", subcore_axis_name="s",
                                  num_cores=NC, num_subcores=NS)


def _sc_params():
    return pltpu.CompilerParams(use_tc_tiling_on_sc=False)


def _gather_body(table_hbm, idx_hbm, out_hbm, idx_v, rows_v, sem):
    wid = lax.axis_index("c") * NS + lax.axis_index("s")
    pltpu.sync_copy(idx_hbm.at[pl.ds(wid * NCH, NCH)], idx_v)

    def fire(j, carry):
        pltpu.async_copy(table_hbm.at[idx_v.at[j]],
                         rows_v.at[pl.ds(j * CHUNK, CHUNK)], sem)
        return carry

    lax.fori_loop(0, NCH, fire, 0)

    def drain(j, carry):
        pltpu.make_async_copy(table_hbm.at[idx_v.at[j]],
                              rows_v.at[pl.ds(j * CHUNK, CHUNK)], sem).wait()
        return carry

    lax.fori_loop(0, NCH, drain, 0)
    pltpu.sync_copy(rows_v, out_hbm.at[pl.ds(wid * EPT, EPT)])


@functools.cache
def _sc_gather():
    return pl.kernel(
        _gather_body,
        out_type=jax.ShapeDtypeStruct((E_PAD, F), jnp.float32),
        mesh=_sc_mesh(),
        compiler_params=_sc_params(),
        scratch_types=[
            pltpu.VMEM((NCH, CHUNK), jnp.int32),
            pltpu.VMEM((EPT, F), jnp.float32),
            pltpu.SemaphoreType.DMA,
        ],
    )


NPT = N_ACC // NS
NHP = NPT // 2


def _mega0_body(msg_hbm, didx_hbm, sidx_hbm, root_hbm,
                xs_out, h1_out, c_out,
                idx_v, buf_v, ones_v, sv, cv, rootv, h1v,
                acc_s, acc_c, sem):
    h1_spm = acc_s
    cid = lax.axis_index("c")
    sid = lax.axis_index("s")
    wid = cid * NS + sid
    owns = (sid < 8) == (cid == 0)

    def zfill(i, carry):
        h1v[i] = jnp.zeros((F,), jnp.float32)
        return carry

    lax.fori_loop(0, NHP, zfill, 0)

    def ofill(i, carry):
        ones_v[i] = jnp.ones((F,), jnp.float32)
        return carry

    lax.fori_loop(0, CHUNK, ofill, 0)
    for q in range(2):
        pltpu.sync_copy(h1v, acc_s.at[pl.ds(sid * NPT + q * NHP, NHP)])
        pltpu.sync_copy(h1v, acc_c.at[pl.ds(sid * NPT + q * NHP, NHP)])
    plsc.subcore_barrier()

    for p in range(2):
        ebase = (sid * 2 + p) * EPT
        pltpu.sync_copy(didx_hbm.at[pl.ds((sid * 2 + p) * NCH, NCH)], idx_v)
        pltpu.sync_copy(msg_hbm.at[pl.ds(ebase, EPT)], buf_v)

        def fire(j, carry):
            pltpu.async_copy(buf_v.at[pl.ds(j * CHUNK, CHUNK)],
                             acc_s.at[idx_v.at[j]], sem, add=True)
            pltpu.async_copy(ones_v, acc_c.at[idx_v.at[j]], sem, add=True)
            return carry

        lax.fori_loop(0, NCH, fire, 0)

        def drain(j, carry):
            pltpu.make_async_copy(buf_v.at[pl.ds(j * CHUNK, CHUNK)],
                                  acc_s.at[idx_v.at[j]], sem).wait()
            pltpu.make_async_copy(ones_v, acc_c.at[idx_v.at[j]], sem).wait()
            return carry

        lax.fori_loop(0, NCH, drain, 0)
    plsc.subcore_barrier()

    for q in range(2):
        nbase = sid * NPT + q * NHP
        pltpu.sync_copy(acc_s.at[pl.ds(nbase, NHP)], sv)
        pltpu.sync_copy(acc_c.at[pl.ds(nbase, NHP)], cv)
        pltpu.sync_copy(root_hbm.at[pl.ds(nbase, NHP)], rootv)

        def node(i, carry):
            t = sv[i] / jnp.maximum(cv[i], 1.0) + rootv[i]
            h1v[i] = jnp.where(t > 0.0, t,
                               jnp.exp(jnp.minimum(t, 0.0)) - 1.0)
            return carry

        lax.fori_loop(0, NHP, node, 0)
        pltpu.sync_copy(h1v, h1_spm.at[pl.ds(nbase, NHP)])

        @pl.when(owns)
        def _():
            pltpu.sync_copy(h1v, h1_out.at[pl.ds(nbase, NHP)])
            pltpu.sync_copy(cv, c_out.at[pl.ds(nbase, NHP)])

    plsc.subcore_barrier()

    pltpu.sync_copy(sidx_hbm.at[pl.ds(wid * NCH, NCH)], idx_v)

    def gfire(j, carry):
        pltpu.async_copy(h1_spm.at[idx_v.at[j]],
                         buf_v.at[pl.ds(j * CHUNK, CHUNK)], sem)
        return carry

    lax.fori_loop(0, NCH, gfire, 0)

    def gdrain(j, carry):
        pltpu.make_async_copy(h1_spm.at[idx_v.at[j]],
                              buf_v.at[pl.ds(j * CHUNK, CHUNK)], sem).wait()
        return carry

    lax.fori_loop(0, NCH, gdrain, 0)
    pltpu.sync_copy(buf_v, xs_out.at[pl.ds(wid * EPT, EPT)])


@functools.cache
def _sc_mega0():
    return pl.kernel(
        _mega0_body,
        out_type=(jax.ShapeDtypeStruct((E_PAD, F), jnp.float32),
                  jax.ShapeDtypeStruct((N_ACC, F), jnp.float32),
                  jax.ShapeDtypeStruct((N_ACC, F), jnp.float32)),
        mesh=_sc_mesh(),
        compiler_params=_sc_params(),
        scratch_types=[
            pltpu.VMEM((NCH, CHUNK), jnp.int32),
            pltpu.VMEM((EPT, F), jnp.float32),
            pltpu.VMEM((CHUNK, F), jnp.float32),
            pltpu.VMEM((NHP, F), jnp.float32),
            pltpu.VMEM((NHP, F), jnp.float32),
            pltpu.VMEM((NHP, F), jnp.float32),
            pltpu.VMEM((NHP, F), jnp.float32),
            pltpu.VMEM_SHARED((N_ACC, F), jnp.float32),
            pltpu.VMEM_SHARED((N_ACC, F), jnp.float32),
            pltpu.SemaphoreType.DMA,
        ],
    )


def _mega1_body(msg_hbm, didx_hbm, root_hbm, c_hbm,
                h2_out,
                idx_v, buf_v, sv, cv, rootv, h1v,
                acc_s, sem):
    cid = lax.axis_index("c")
    sid = lax.axis_index("s")
    owns = (sid < 8) == (cid == 0)

    def zfill(i, carry):
        h1v[i] = jnp.zeros((F,), jnp.float32)
        return carry

    lax.fori_loop(0, NHP, zfill, 0)
    for q in range(2):
        pltpu.sync_copy(h1v, acc_s.at[pl.ds(sid * NPT + q * NHP, NHP)])
    plsc.subcore_barrier()

    for p in range(2):
        ebase = (sid * 2 + p) * EPT
        pltpu.sync_copy(didx_hbm.at[pl.ds((sid * 2 + p) * NCH, NCH)], idx_v)
        pltpu.sync_copy(msg_hbm.at[pl.ds(ebase, EPT)], buf_v)

        def fire(j, carry):
            pltpu.async_copy(buf_v.at[pl.ds(j * CHUNK, CHUNK)],
                             acc_s.at[idx_v.at[j]], sem, add=True)
            return carry

        lax.fori_loop(0, NCH, fire, 0)

        def drain(j, carry):
            pltpu.make_async_copy(buf_v.at[pl.ds(j * CHUNK, CHUNK)],
                                  acc_s.at[idx_v.at[j]], sem).wait()
            return carry

        lax.fori_loop(0, NCH, drain, 0)
    plsc.subcore_barrier()

    for q in range(2):
        nbase = sid * NPT + q * NHP
        pltpu.sync_copy(acc_s.at[pl.ds(nbase, NHP)], sv)
        pltpu.sync_copy(c_hbm.at[pl.ds(nbase, NHP)], cv)
        pltpu.sync_copy(root_hbm.at[pl.ds(nbase, NHP)], rootv)

        def node(i, carry):
            t = sv[i] / jnp.maximum(cv[i], 1.0) + rootv[i]
            h1v[i] = jnp.where(t > 0.0, t,
                               jnp.exp(jnp.minimum(t, 0.0)) - 1.0)
            return carry

        lax.fori_loop(0, NHP, node, 0)

        @pl.when(owns)
        def _():
            pltpu.sync_copy(h1v, h2_out.at[pl.ds(nbase, NHP)])


@functools.cache
def _sc_mega1():
    return pl.kernel(
        _mega1_body,
        out_type=jax.ShapeDtypeStruct((N_ACC, F), jnp.float32),
        mesh=_sc_mesh(),
        compiler_params=_sc_params(),
        scratch_types=[
            pltpu.VMEM((NCH, CHUNK), jnp.int32),
            pltpu.VMEM((EPT, F), jnp.float32),
            pltpu.VMEM((NHP, F), jnp.float32),
            pltpu.VMEM((NHP, F), jnp.float32),
            pltpu.VMEM((NHP, F), jnp.float32),
            pltpu.VMEM((NHP, F), jnp.float32),
            pltpu.VMEM_SHARED((N_ACC, F), jnp.float32),
            pltpu.SemaphoreType.DMA,
        ],
    )


def _scatter_counts_body(msg_hbm, idx_hbm, s_out, c_out,
                         idx_v, msg_v, zbuf, acc_s, ones_v, acc_c, sem):
    cid = lax.axis_index("c")
    sid = lax.axis_index("s")
    wid = cid * NS + sid

    def zfill(i, carry):
        zbuf[i] = jnp.zeros((F,), jnp.float32)
        return carry

    lax.fori_loop(0, RPT, zfill, 0)

    def ofill(i, carry):
        ones_v[i] = jnp.ones((F,), jnp.float32)
        return carry

    lax.fori_loop(0, CHUNK, ofill, 0)
    pltpu.sync_copy(zbuf, acc_s.at[pl.ds(sid * RPT, RPT)])
    pltpu.sync_copy(zbuf, acc_c.at[pl.ds(sid * RPT, RPT)])
    pltpu.sync_copy(idx_hbm.at[pl.ds(wid * NCH, NCH)], idx_v)
    pltpu.sync_copy(msg_hbm.at[pl.ds(wid * EPT, EPT)], msg_v)
    plsc.subcore_barrier()

    def fire(j, carry):
        pltpu.async_copy(msg_v.at[pl.ds(j * CHUNK, CHUNK)],
                         acc_s.at[idx_v.at[j]], sem, add=True)
        return carry

    lax.fori_loop(0, NCH, fire, 0)

    def drain(j, carry):
        pltpu.make_async_copy(msg_v.at[pl.ds(j * CHUNK, CHUNK)],
                              acc_s.at[idx_v.at[j]], sem).wait()
        return carry

    lax.fori_loop(0, NCH, drain, 0)

    def fire_c(j, carry):
        pltpu.async_copy(ones_v, acc_c.at[idx_v.at[j]], sem, add=True)
        return carry

    lax.fori_loop(0, NCH, fire_c, 0)

    def drain_c(j, carry):
        pltpu.make_async_copy(ones_v, acc_c.at[idx_v.at[j]], sem).wait()
        return carry

    lax.fori_loop(0, NCH, drain_c, 0)
    plsc.subcore_barrier()
    pltpu.sync_copy(acc_s.at[pl.ds(sid * RPT, RPT)],
                    s_out.at[cid, pl.ds(sid * RPT, RPT)])
    pltpu.sync_copy(acc_c.at[pl.ds(sid * RPT, RPT)],
                    c_out.at[cid, pl.ds(sid * RPT, RPT)])


@functools.cache
def _sc_scatter_counts():
    return pl.kernel(
        _scatter_counts_body,
        out_type=(jax.ShapeDtypeStruct((NC, N_ACC, F), jnp.float32),
                  jax.ShapeDtypeStruct((NC, N_ACC, F), jnp.float32)),
        mesh=_sc_mesh(),
        compiler_params=_sc_params(),
        scratch_types=[
            pltpu.VMEM((NCH, CHUNK), jnp.int32),
            pltpu.VMEM((EPT, F), jnp.float32),
            pltpu.VMEM((RPT, F), jnp.float32),
            pltpu.VMEM_SHARED((N_ACC, F), jnp.float32),
            pltpu.VMEM((CHUNK, F), jnp.float32),
            pltpu.VMEM_SHARED((N_ACC, F), jnp.float32),
            pltpu.SemaphoreType.DMA,
        ],
    )


def _scatter_body(msg_hbm, idx_hbm, s_out, idx_v, msg_v, zbuf, acc_s, sem):
    cid = lax.axis_index("c")
    sid = lax.axis_index("s")
    wid = cid * NS + sid

    def zfill(i, carry):
        zbuf[i] = jnp.zeros((F,), jnp.float32)
        return carry

    lax.fori_loop(0, RPT, zfill, 0)
    pltpu.sync_copy(zbuf, acc_s.at[pl.ds(sid * RPT, RPT)])
    pltpu.sync_copy(idx_hbm.at[pl.ds(wid * NCH, NCH)], idx_v)
    pltpu.sync_copy(msg_hbm.at[pl.ds(wid * EPT, EPT)], msg_v)
    plsc.subcore_barrier()

    def fire(j, carry):
        pltpu.async_copy(msg_v.at[pl.ds(j * CHUNK, CHUNK)],
                         acc_s.at[idx_v.at[j]], sem, add=True)
        return carry

    lax.fori_loop(0, NCH, fire, 0)

    def drain(j, carry):
        pltpu.make_async_copy(msg_v.at[pl.ds(j * CHUNK, CHUNK)],
                              acc_s.at[idx_v.at[j]], sem).wait()
        return carry

    lax.fori_loop(0, NCH, drain, 0)
    plsc.subcore_barrier()
    pltpu.sync_copy(acc_s.at[pl.ds(sid * RPT, RPT)],
                    s_out.at[cid, pl.ds(sid * RPT, RPT)])


@functools.cache
def _sc_scatter():
    return pl.kernel(
        _scatter_body,
        out_type=jax.ShapeDtypeStruct((NC, N_ACC, F), jnp.float32),
        mesh=_sc_mesh(),
        compiler_params=_sc_params(),
        scratch_types=[
            pltpu.VMEM((NCH, CHUNK), jnp.int32),
            pltpu.VMEM((EPT, F), jnp.float32),
            pltpu.VMEM((RPT, F), jnp.float32),
            pltpu.VMEM_SHARED((N_ACC, F), jnp.float32),
            pltpu.SemaphoreType.DMA,
        ],
    )


def _msg_body(ea_ref, xs_ref, hp_ref, w1_ref, b1_ref, w2_ref, e1_ref, r_ref,
              wr_ref, brt_ref, out_ref, root_ref):
    h = jnp.dot(ea_ref[...], w1_ref[...], preferred_element_type=jnp.float32)
    h = jnp.maximum(h + b1_ref[...], 0.0)
    w = jnp.dot(h.astype(jnp.bfloat16), w2_ref[...],
                preferred_element_type=jnp.float32)
    xe = jnp.dot(xs_ref[...].astype(jnp.bfloat16), e1_ref[...],
                 preferred_element_type=jnp.float32)
    out_ref[...] = jnp.dot((w * xe).astype(jnp.bfloat16), r_ref[...],
                           preferred_element_type=jnp.float32)
    root_ref[...] = jnp.dot(hp_ref[...], wr_ref[...],
                            preferred_element_type=jnp.float32) + brt_ref[...]


BR = BE // 8
NBR = (N_ACC // 8) // (E_PAD // BE)


def _tc_msg(ea_p, xs_p, hp, w1bd, b1t, w2bd, e1bd, rbd, wrbd, brt):
    grid = (E_PAD // BE,)
    return pl.pallas_call(
        _msg_body,
        grid=grid,
        in_specs=[
            pl.BlockSpec((BR, 64), lambda i: (i, 0)),
            pl.BlockSpec((BR, 128), lambda i: (i, 0)),
            pl.BlockSpec((NBR, 128), lambda i: (i, 0)),
            pl.BlockSpec((64, 256), lambda i: (0, 0)),
            pl.BlockSpec((1, 256), lambda i: (0, 0)),
            pl.BlockSpec((256, 2048), lambda i: (0, 0)),
            pl.BlockSpec((128, 2048), lambda i: (0, 0)),
            pl.BlockSpec((2048, 128), lambda i: (0, 0)),
            pl.BlockSpec((128, 128), lambda i: (0, 0)),
            pl.BlockSpec((1, 128), lambda i: (0, 0)),
        ],
        out_specs=[
            pl.BlockSpec((BR, 128), lambda i: (i, 0)),
            pl.BlockSpec((NBR, 128), lambda i: (i, 0)),
        ],
        out_shape=[
            jax.ShapeDtypeStruct((E_PAD // 8, 128), jnp.float32),
            jax.ShapeDtypeStruct((N_ACC // 8, 128), jnp.float32),
        ],
    )(ea_p, xs_p, hp, w1bd, b1t, w2bd, e1bd, rbd, wrbd, brt)


BNP = BN // 8


def _node_body(s_ref, c_ref, h_ref, wr_ref, br_ref, out_ref):
    s = s_ref[0] + s_ref[1]
    c = c_ref[0] + c_ref[1]
    agg = s / jnp.maximum(c, 1.0)
    t = agg + jnp.dot(h_ref[...], wr_ref[...],
                      preferred_element_type=jnp.float32) + br_ref[...]
    out_ref[...] = jnp.where(t > 0.0, t, jnp.exp(jnp.minimum(t, 0.0)) - 1.0)


def _tc_node(s_part, c_part, h_prev_p, wrbd, brt):
    grid = (N_ACC // BN,)
    return pl.pallas_call(
        _node_body,
        grid=grid,
        in_specs=[
            pl.BlockSpec((NC, BNP, 128), lambda i: (0, i, 0)),
            pl.BlockSpec((NC, BNP, 128), lambda i: (0, i, 0)),
            pl.BlockSpec((BNP, 128), lambda i: (i, 0)),
            pl.BlockSpec((128, 128), lambda i: (0, 0)),
            pl.BlockSpec((1, 128), lambda i: (0, 0)),
        ],
        out_specs=pl.BlockSpec((BNP, 128), lambda i: (i, 0)),
        out_shape=jax.ShapeDtypeStruct((N_ACC // 8, 128), jnp.float32),
    )(s_part, c_part, h_prev_p, wrbd, brt)


def _pool_body(h_ref, b_ref, wf_ref, bf_ref, out_ref, acc, cnt):
    i = pl.program_id(0)

    @pl.when(i == 0)
    def _():
        acc[...] = jnp.zeros_like(acc)
        cnt[...] = jnp.zeros_like(cnt)

    gid = lax.broadcasted_iota(jnp.int32, (NUM_GRAPHS, BN), 0)
    oh = (gid == b_ref[0]).astype(jnp.float32)
    acc[...] += jnp.dot(oh, h_ref[...], preferred_element_type=jnp.float32)
    cnt[...] += jnp.dot(oh, jnp.ones((BN, F), jnp.float32),
                        preferred_element_type=jnp.float32)
    pooled = acc[...] / jnp.maximum(cnt[...], 1.0)
    out_ref[...] = jnp.dot(pooled, wf_ref[...],
                           preferred_element_type=jnp.float32) + bf_ref[...]


def _tc_pool(h2, batch3d, wf, bf):
    grid = (N_ACC // BN,)
    return pl.pallas_call(
        _pool_body,
        grid=grid,
        in_specs=[
            pl.BlockSpec((BN, F), lambda i: (i, 0)),
            pl.BlockSpec((1, 1, BN), lambda i: (i, 0, 0)),
            pl.BlockSpec((F, C_OUT), lambda i: (0, 0)),
            pl.BlockSpec((1, C_OUT), lambda i: (0, 0)),
        ],
        out_specs=pl.BlockSpec((NUM_GRAPHS, C_OUT), lambda i: (0, 0)),
        out_shape=jax.ShapeDtypeStruct((NUM_GRAPHS, C_OUT), jnp.float32),
        scratch_shapes=[
            pltpu.VMEM((NUM_GRAPHS, F), jnp.float32),
            pltpu.VMEM((NUM_GRAPHS, F), jnp.float32),
        ],
    )(h2, batch3d, wf, bf)


def _fold_layer(p, ey8):
    g = p['gamma'] / jnp.sqrt(1.0 + 1e-5)
    w1f = p['W1'] * g[None, :]
    b1f = p['b1'] * g + p['beta']
    w1e = jnp.zeros((8, 32), jnp.float32).at[:D_EDGE, :HID].set(w1f)
    b1e = jnp.zeros((1, 32), jnp.float32).at[0, :HID].set(b1f).at[0, HID].set(1.0)
    w2e = jnp.zeros((32, 256), jnp.float32).at[:HID].set(p['W2']).at[HID].set(p['b2'])
    w1bd = jnp.kron(ey8, w1e)
    b1t = jnp.tile(b1e, (1, 8))
    w2bd = jnp.kron(ey8.astype(jnp.bfloat16),
                    w2e.astype(jnp.bfloat16))
    wrbd = jnp.kron(ey8, p['Wr'])
    brt = jnp.tile(p['br'][None, :], (1, 8))
    return w1bd, b1t, w2bd, wrbd, brt


def kernel(x, edge_index, edge_attr, batch, params):
    src = edge_index[0]
    dst = edge_index[1]
    pad = E_PAD - E
    src2d = jnp.concatenate(
        [src, jnp.zeros((pad,), jnp.int32)]).reshape(NW * NCH, CHUNK)
    dst2d = jnp.concatenate(
        [dst, jnp.full((pad,), N, jnp.int32)]).reshape(NW * NCH, CHUNK)
    ea_p = jnp.zeros((E_PAD, 8), jnp.float32).at[:E, :D_EDGE].set(
        edge_attr).reshape(E_PAD // 8, 64)

    ey8 = jnp.eye(8, dtype=jnp.float32)
    ey8b = jnp.eye(8, dtype=jnp.bfloat16)
    e1bd = jnp.kron(ey8b, jnp.kron(jnp.eye(F, dtype=jnp.bfloat16),
                                   jnp.ones((1, F), jnp.bfloat16)))
    rbd = jnp.kron(ey8b, jnp.kron(jnp.ones((F, 1), jnp.bfloat16),
                                  jnp.eye(F, dtype=jnp.bfloat16)))

    w1bd0, b1t0, w2bd0, wrbd0, brt0 = _fold_layer(params['conv0'], ey8)
    w1bd1, b1t1, w2bd1, wrbd1, brt1 = _fold_layer(params['conv1'], ey8)

    x_p = jnp.zeros((N_ACC, F), jnp.float32).at[:N].set(x).reshape(
        N_ACC // 8, 128)
    batch3d = jnp.concatenate(
        [batch, jnp.full((N_ACC - N,), NUM_GRAPHS, jnp.int32)]).reshape(
        N_ACC // BN, 1, BN)

    xs0 = _sc_gather()(x, src2d)
    msg0, root0 = _tc_msg(ea_p, xs0.reshape(E_PAD // 8, 128), x_p,
                          w1bd0, b1t0, w2bd0, e1bd, rbd, wrbd0, brt0)
    xs1, h1, cfull = _sc_mega0()(msg0.reshape(E_PAD, F), dst2d, src2d,
                                 root0.reshape(N_ACC, F))
    msg1, root1 = _tc_msg(ea_p, xs1.reshape(E_PAD // 8, 128),
                          h1.reshape(N_ACC // 8, 128),
                          w1bd1, b1t1, w2bd1, e1bd, rbd, wrbd1, brt1)
    h2 = _sc_mega1()(msg1.reshape(E_PAD, F), dst2d,
                     root1.reshape(N_ACC, F), cfull)
    return _tc_pool(h2, batch3d, params['Wf'], params['bf'][None, :])
